# baseline (speedup 1.0000x reference)
import functools

import jax
import jax.numpy as jnp
from jax import lax
from jax.experimental import pallas as pl
from jax.experimental.pallas import tpu as pltpu
from jax.experimental.pallas import tpu_sc as plsc

_NC = 2
_NS = 16
_NW = _NC * _NS
_LANES = 16


def _make_kernel(vocab, dim, batch):
    b_per_w = batch // _NW
    n_groups = b_per_w // _LANES
    gwords = dim * _LANES
    n_tr = gwords // 128
    mesh = plsc.VectorSubcoreMesh(core_axis_name="c", subcore_axis_name="s")

    @functools.partial(
        pl.kernel,
        out_type=jax.ShapeDtypeStruct((batch,), jnp.float32),
        mesh=mesh,
        compiler_params=pltpu.CompilerParams(needs_layout_passes=False),
        scratch_types=[
            pltpu.VMEM((b_per_w,), jnp.int32),
            pltpu.VMEM((b_per_w,), jnp.int32),
            pltpu.VMEM((2 * n_tr, 128), jnp.int32),
            pltpu.VMEM((gwords,), jnp.float32),
            pltpu.VMEM((gwords,), jnp.float32),
            pltpu.VMEM((b_per_w,), jnp.float32),
            pltpu.SemaphoreType.DMA,
        ],
    )
    def k(hero_ids, ctx_ids, hero_flat, ctx_flat, out,
          hidx_v, cidx_v, addr_v, hval_v, cval_v, score_v, sem):
        wid = lax.axis_index("s") * _NC + lax.axis_index("c")
        base = wid * b_per_w

        pltpu.sync_copy(hero_ids.at[pl.ds(base, b_per_w)], hidx_v)
        pltpu.sync_copy(ctx_ids.at[pl.ds(base, b_per_w)], cidx_v)

        def group(g, carry):
            e0 = g * _LANES
            hr = hidx_v[pl.ds(e0, _LANES)]
            cr = cidx_v[pl.ds(e0, _LANES)]
            for d in range(dim):
                row = (d * _LANES) // 128
                col = (d * _LANES) % 128
                addr_v[row, pl.ds(col, _LANES)] = hr + (d * vocab)
                addr_v[n_tr + row, pl.ds(col, _LANES)] = cr + (d * vocab)
            copies = []
            for j in range(n_tr):
                copies.append(pltpu.async_copy(
                    hero_flat.at[addr_v.at[j]],
                    hval_v.at[pl.ds(j * 128, 128)], sem))
                copies.append(pltpu.async_copy(
                    ctx_flat.at[addr_v.at[n_tr + j]],
                    cval_v.at[pl.ds(j * 128, 128)], sem))
            for c in copies:
                c.wait()

            acc = jnp.zeros((_LANES,), jnp.float32)
            for d in range(dim):
                h = hval_v[pl.ds(d * _LANES, _LANES)]
                c = cval_v[pl.ds(d * _LANES, _LANES)]
                acc = acc + h * c
            score_v[pl.ds(e0, _LANES)] = acc
            return carry

        lax.fori_loop(0, n_groups, group, 0)

        pltpu.sync_copy(score_v, out.at[pl.ds(base, b_per_w)])

    return k


@jax.jit
def kernel(hero_ids, context_ids, hero_table, context_table):
    vocab, dim = hero_table.shape
    batch = hero_ids.shape[0]
    k = _make_kernel(vocab, dim, batch)
    hero_flat = hero_table.T.reshape(-1)
    ctx_flat = context_table.T.reshape(-1)
    return k(hero_ids.astype(jnp.int32), context_ids.astype(jnp.int32),
             hero_flat, ctx_flat)


# speedup vs baseline: 12.4258x; 12.4258x over previous
import functools

import jax
import jax.numpy as jnp
from jax import lax
from jax.experimental import pallas as pl
from jax.experimental.pallas import tpu as pltpu
from jax.experimental.pallas import tpu_sc as plsc

_NC = 2
_NS = 16
_NW = _NC * _NS
_LANES = 16


def _make_kernel(vocab, dim, batch):
    b_per_w = batch // _NW
    n_groups = b_per_w // _LANES
    mesh = plsc.VectorSubcoreMesh(core_axis_name="c", subcore_axis_name="s")

    @functools.partial(
        pl.kernel,
        out_type=jax.ShapeDtypeStruct((batch,), jnp.float32),
        mesh=mesh,
        compiler_params=pltpu.CompilerParams(needs_layout_passes=False),
        scratch_types=[
            pltpu.VMEM((b_per_w,), jnp.int32),
            pltpu.VMEM((b_per_w,), jnp.int32),
            pltpu.VMEM((_LANES, 8, dim), jnp.float32),
            pltpu.VMEM((_LANES, 8, dim), jnp.float32),
            pltpu.VMEM((b_per_w,), jnp.float32),
            pltpu.SemaphoreType.DMA,
        ],
    )
    def k(hero_ids, ctx_ids, hero_tab, ctx_tab, out,
          hidx_v, cidx_v, hbuf, cbuf, score_v, sem):
        wid = lax.axis_index("s") * _NC + lax.axis_index("c")
        base = wid * b_per_w

        pltpu.sync_copy(hero_ids.at[pl.ds(base, b_per_w)], hidx_v)
        pltpu.sync_copy(ctx_ids.at[pl.ds(base, b_per_w)], cidx_v)

        lane = lax.iota(jnp.int32, _LANES)

        def group(g, carry):
            e0 = g * _LANES
            hiv = hidx_v[pl.ds(e0, _LANES)]
            civ = cidx_v[pl.ds(e0, _LANES)]
            htile = lax.shift_right_logical(hiv, 3)
            ctile = lax.shift_right_logical(civ, 3)
            copies = []
            for j in range(_LANES):
                copies.append(pltpu.async_copy(
                    hero_tab.at[htile[j]], hbuf.at[j], sem))
                copies.append(pltpu.async_copy(
                    ctx_tab.at[ctile[j]], cbuf.at[j], sem))
            for c in copies:
                c.wait()

            hsub = hiv & 7
            csub = civ & 7
            acc = jnp.zeros((_LANES,), jnp.float32)
            for d in range(dim):
                col = jnp.full((_LANES,), d, jnp.int32)
                h = plsc.load_gather(hbuf, [lane, hsub, col])
                c = plsc.load_gather(cbuf, [lane, csub, col])
                acc = acc + h * c
            score_v[pl.ds(e0, _LANES)] = acc
            return carry

        lax.fori_loop(0, n_groups, group, 0)

        pltpu.sync_copy(score_v, out.at[pl.ds(base, b_per_w)])

    return k


@jax.jit
def kernel(hero_ids, context_ids, hero_table, context_table):
    vocab, dim = hero_table.shape
    batch = hero_ids.shape[0]
    k = _make_kernel(vocab, dim, batch)
    hero3 = hero_table.reshape(vocab // 8, 8, dim)
    ctx3 = context_table.reshape(vocab // 8, 8, dim)
    return k(hero_ids.astype(jnp.int32), context_ids.astype(jnp.int32),
             hero3, ctx3)
